# baseline (speedup 1.0000x reference)
import jax
from jax.experimental import pallas as pl
from jax.experimental.pallas import tpu as pltpu

_NCHUNKS = 8


def _copy_dma(x_ref, o_ref):
    rows = x_ref.shape[0] // _NCHUNKS

    def body(*sems):
        copies = [
            pltpu.make_async_copy(
                x_ref.at[pl.ds(i * rows, rows)],
                o_ref.at[pl.ds(i * rows, rows)],
                sems[i],
            )
            for i in range(_NCHUNKS)
        ]
        for c in copies:
            c.start()
        for c in copies:
            c.wait()

    pl.run_scoped(body, *([pltpu.SemaphoreType.DMA] * _NCHUNKS))


def kernel(data, partitions):
    del partitions
    return pl.pallas_call(
        _copy_dma,
        in_specs=[pl.BlockSpec(memory_space=pl.ANY)],
        out_specs=pl.BlockSpec(memory_space=pl.ANY),
        out_shape=jax.ShapeDtypeStruct(data.shape, data.dtype),
    )(data)


# speedup vs baseline: 16.3012x; 16.3012x over previous
import jax
from jax.experimental import pallas as pl

_BLOCK_ROWS = 16384


def _copy_block(x_ref, o_ref):
    o_ref[...] = x_ref[...]


def kernel(data, partitions):
    del partitions
    n, d = data.shape
    return pl.pallas_call(
        _copy_block,
        grid=(n // _BLOCK_ROWS,),
        in_specs=[pl.BlockSpec((_BLOCK_ROWS, d), lambda i: (i, 0))],
        out_specs=pl.BlockSpec((_BLOCK_ROWS, d), lambda i: (i, 0)),
        out_shape=jax.ShapeDtypeStruct((n, d), data.dtype),
    )(data)
